# Initial kernel scaffold; baseline (speedup 1.0000x reference)
import jax
import jax.numpy as jnp
from jax.experimental import pallas as pl


def kernel(proposals, gt_bboxes, scores, nms_pre):
    raise NotImplementedError("write your pallas kernel here")


# speedup vs baseline: 28.1387x; 28.1387x over previous
import functools

import jax
import jax.numpy as jnp
from jax import lax
from jax.experimental import pallas as pl
from jax.experimental.pallas import tpu as pltpu

N = 5000
P = 5120
B = 128
NB = P // B
IOU_THR = 0.5
NMS_THR = 0.7


def _iou(ax1, ay1, ax2, ay2, bx1, by1, bx2, by2):
    area_a = (ax2 - ax1) * (ay2 - ay1)
    area_b = (bx2 - bx1) * (by2 - by1)
    w = jnp.maximum(jnp.minimum(ax2, bx2) - jnp.maximum(ax1, bx1), 0.0)
    h = jnp.maximum(jnp.minimum(ay2, by2) - jnp.maximum(ay1, by1), 0.0)
    inter = w * h
    union = area_a + area_b - inter
    return inter / jnp.maximum(union, 1e-6)


def _stage_a(gt_ref, np_ref, pr_ref, pc_ref, eff_ref, rank_ref):
    gx1 = gt_ref[0]
    gy1 = gt_ref[1]
    gx2 = gt_ref[2]
    gy2 = gt_ref[3]
    kf = np_ref[0]

    x1r = pr_ref[0:1, :]
    y1r = pr_ref[1:2, :]
    x2r = pr_ref[2:3, :]
    y2r = pr_ref[3:4, :]
    scr = pr_ref[4:5, :]
    lane = lax.broadcasted_iota(jnp.int32, (1, P), 1)
    valid_r = lane < N
    ovr = _iou(gx1, gy1, gx2, gy2, x1r, y1r, x2r, y2r)
    ovr = jnp.where(valid_r, ovr, -1.0)

    x1c = pc_ref[:, 0:1]
    y1c = pc_ref[:, 1:2]
    x2c = pc_ref[:, 2:3]
    y2c = pc_ref[:, 3:4]
    scc = pc_ref[:, 4:5]
    subl = lax.broadcasted_iota(jnp.int32, (P, 1), 0)
    valid_c = subl < N
    ovc = _iou(gx1, gy1, gx2, gy2, x1c, y1c, x2c, y2c)
    ovc = jnp.where(valid_c, ovc, -1.0)

    posr = (ovr > IOU_THR).astype(jnp.float32)
    hp = jnp.max(posr, axis=(0, 1), keepdims=True)

    cacc = jnp.zeros((1, P), jnp.float32)
    for jb in range(NB):
        ovj = ovc[jb * B:(jb + 1) * B, :]
        cmp = (ovj > ovr).astype(jnp.float32)
        cacc = cacc + jnp.sum(cmp, axis=0, keepdims=True)
    kth = jnp.min(jnp.where(cacc < kf, ovr, jnp.inf), axis=(0, 1),
                  keepdims=True)

    fbr = (ovr >= kth).astype(jnp.float32)
    fmr = hp * posr + (1.0 - hp) * fbr
    effr = jnp.where((fmr > 0.5) & valid_r, scr, -1.0)

    posc = (ovc > IOU_THR).astype(jnp.float32)
    fbc = (ovc >= kth).astype(jnp.float32)
    fmc = hp * posc + (1.0 - hp) * fbc
    effc = jnp.where((fmc > 0.5) & valid_c, scc, -1.0)

    racc = jnp.zeros((1, P), jnp.float32)
    for jb in range(NB):
        ej = effc[jb * B:(jb + 1) * B, :]
        ji = jb * B + lax.broadcasted_iota(jnp.int32, (B, 1), 0)
        cmp = (ej > effr) | ((ej == effr) & (ji < lane))
        racc = racc + jnp.sum(cmp.astype(jnp.float32), axis=0, keepdims=True)

    eff_ref[...] = effc
    rank_ref[...] = racc


def _stage_c(rank_ref, dc_ref, dr_ref, out_ref, sc_ref, sr_ref, keep_ref,
             m_ref):
    rank = rank_ref[...]
    for rb in range(NB):
        rid = (rb * B + lax.broadcasted_iota(jnp.int32, (B, 1), 0)
               ).astype(jnp.float32)
        ph = (rank == rid).astype(jnp.float32)
        blk_c = lax.dot_general(ph, dc_ref[...],
                                (((1,), (0,)), ((), ())),
                                preferred_element_type=jnp.float32,
                                precision=lax.Precision.HIGHEST)
        sc_ref[rb * B:(rb + 1) * B, :] = blk_c
        blk_r = lax.dot_general(dr_ref[...], ph,
                                (((1,), (1,)), ((), ())),
                                preferred_element_type=jnp.float32,
                                precision=lax.Precision.HIGHEST)
        sr_ref[:, rb * B:(rb + 1) * B] = blk_r

    keep_ref[...] = (sr_ref[4:5, :] > 0.0).astype(jnp.float32)

    lane_b = lax.broadcasted_iota(jnp.int32, (1, B), 1)
    sub_b = lax.broadcasted_iota(jnp.int32, (B, 1), 0)
    ident = (lane_b == sub_b).astype(jnp.float32)

    for bi in range(NB):
        s0 = bi * B
        xi1 = sc_ref[s0:s0 + B, 0:1]
        yi1 = sc_ref[s0:s0 + B, 1:2]
        xi2 = sc_ref[s0:s0 + B, 2:3]
        yi2 = sc_ref[s0:s0 + B, 3:4]
        xj1 = sr_ref[0:1, s0:s0 + B]
        yj1 = sr_ref[1:2, s0:s0 + B]
        xj2 = sr_ref[2:3, s0:s0 + B]
        yj2 = sr_ref[3:4, s0:s0 + B]
        iou_d = _iou(xi1, yi1, xi2, yi2, xj1, yj1, xj2, yj2)
        m_ref[...] = ((iou_d > NMS_THR) & (lane_b > sub_b)).astype(
            jnp.float32)

        kb0 = keep_ref[:, s0:s0 + B]

        def body(j, kb):
            mj = m_ref[pl.ds(j, 1), :]
            oh = (lane_b == j).astype(jnp.float32)
            kj = jnp.sum(kb * oh, axis=(0, 1), keepdims=True)
            return kb * (1.0 - mj * kj)

        kb = lax.fori_loop(0, B, body, kb0)
        keep_ref[:, s0:s0 + B] = kb

        rest = P - (bi + 1) * B
        if rest > 0:
            kbc = lax.dot_general(ident, kb, (((1,), (1,)), ((), ())),
                                  preferred_element_type=jnp.float32,
                                precision=lax.Precision.HIGHEST)
            t0 = (bi + 1) * B
            xt1 = sr_ref[0:1, t0:t0 + rest]
            yt1 = sr_ref[1:2, t0:t0 + rest]
            xt2 = sr_ref[2:3, t0:t0 + rest]
            yt2 = sr_ref[3:4, t0:t0 + rest]
            iou_x = _iou(xi1, yi1, xi2, yi2, xt1, yt1, xt2, yt2)
            sup = jnp.max((iou_x > NMS_THR).astype(jnp.float32) * kbc,
                          axis=0, keepdims=True)
            keep_ref[:, t0:t0 + rest] = keep_ref[:, t0:t0 + rest] * (1.0 - sup)

    out_ref[...] = sr_ref[...] * keep_ref[...]


@jax.jit
def kernel(proposals, gt_bboxes, scores, nms_pre):
    prop = jnp.asarray(proposals, jnp.float32)
    sc = jnp.asarray(scores, jnp.float32)
    prop_p = jnp.concatenate([prop, jnp.zeros((P - N, 4), jnp.float32)], 0)
    sc_p = jnp.concatenate([sc, jnp.zeros((P - N,), jnp.float32)], 0)
    pc = jnp.concatenate(
        [prop_p, sc_p[:, None], jnp.zeros((P, 3), jnp.float32)], 1)
    pr = pc.T
    gt0 = gt_bboxes[0].astype(jnp.float32)
    npre = jnp.asarray(nms_pre, jnp.float32).reshape((1,))

    eff_c, rank_r = pl.pallas_call(
        _stage_a,
        out_shape=[
            jax.ShapeDtypeStruct((P, 1), jnp.float32),
            jax.ShapeDtypeStruct((1, P), jnp.float32),
        ],
        in_specs=[
            pl.BlockSpec(memory_space=pltpu.SMEM),
            pl.BlockSpec(memory_space=pltpu.SMEM),
            pl.BlockSpec(memory_space=pltpu.VMEM),
            pl.BlockSpec(memory_space=pltpu.VMEM),
        ],
        out_specs=[
            pl.BlockSpec(memory_space=pltpu.VMEM),
            pl.BlockSpec(memory_space=pltpu.VMEM),
        ],
    )(gt0, npre, pr, pc)

    data_c = jnp.concatenate([prop_p, eff_c, jnp.zeros((P, 3), jnp.float32)],
                             1)
    data_r = data_c.T

    dets_t = pl.pallas_call(
        _stage_c,
        out_shape=jax.ShapeDtypeStruct((8, P), jnp.float32),
        scratch_shapes=[
            pltpu.VMEM((P, 8), jnp.float32),
            pltpu.VMEM((8, P), jnp.float32),
            pltpu.VMEM((1, P), jnp.float32),
            pltpu.VMEM((B, B), jnp.float32),
        ],
    )(rank_r, data_c, data_r)

    return dets_t.T[:N, :5]


# speedup vs baseline: 109.6660x; 3.8973x over previous
import jax
import jax.numpy as jnp
from jax import lax
from jax.experimental import pallas as pl
from jax.experimental.pallas import tpu as pltpu

N = 5000
P = 5120
B = 128
NB = P // B
IOU_THR = 0.5
NMS_THR = 0.7
_ONE_BITS = 0x3F800000


def _iou(ax1, ay1, ax2, ay2, bx1, by1, bx2, by2):
    area_a = (ax2 - ax1) * (ay2 - ay1)
    area_b = (bx2 - bx1) * (by2 - by1)
    w = jnp.maximum(jnp.minimum(ax2, bx2) - jnp.maximum(ax1, bx1), 0.0)
    h = jnp.maximum(jnp.minimum(ay2, by2) - jnp.maximum(ay1, by1), 0.0)
    inter = w * h
    union = area_a + area_b - inter
    return inter / jnp.maximum(union, 1e-6)


def _body(gt_ref, np_ref, pr_ref, pc_ref, out_ref, sc_ref, sr_ref,
          keep_ref, m_ref):
    gx1 = gt_ref[0]
    gy1 = gt_ref[1]
    gx2 = gt_ref[2]
    gy2 = gt_ref[3]
    kf = np_ref[0]

    x1r = pr_ref[0:1, :]
    y1r = pr_ref[1:2, :]
    x2r = pr_ref[2:3, :]
    y2r = pr_ref[3:4, :]
    scr = pr_ref[4:5, :]
    lane = lax.broadcasted_iota(jnp.int32, (1, P), 1)
    valid_r = lane < N
    ovr = _iou(gx1, gy1, gx2, gy2, x1r, y1r, x2r, y2r)
    ovr = jnp.where(valid_r, ovr, -1.0)

    x1c = pc_ref[:, 0:1]
    y1c = pc_ref[:, 1:2]
    x2c = pc_ref[:, 2:3]
    y2c = pc_ref[:, 3:4]
    scc = pc_ref[:, 4:5]
    subl = lax.broadcasted_iota(jnp.int32, (P, 1), 0)
    valid_c = subl < N
    ovc = _iou(gx1, gy1, gx2, gy2, x1c, y1c, x2c, y2c)
    ovc = jnp.where(valid_c, ovc, -1.0)

    posr = (ovr > IOU_THR).astype(jnp.float32)
    hp = jnp.max(posr)

    def _bis(_, carry):
        lo, hi = carry
        mid = (lo + hi) // 2
        midv = lax.bitcast_convert_type(jnp.maximum(mid, 0), jnp.float32)
        midf = jnp.where(mid < 0, -0.5, midv)
        g = jnp.sum((ovr > midf).astype(jnp.float32))
        pred = g < kf
        return (jnp.where(pred, lo, mid), jnp.where(pred, mid, hi))

    _, hi = lax.fori_loop(0, 31, _bis,
                          (jnp.int32(-1), jnp.int32(_ONE_BITS)))
    kth = lax.bitcast_convert_type(hi, jnp.float32)

    fbr = (ovr >= kth).astype(jnp.float32)
    fmr = hp * posr + (1.0 - hp) * fbr
    effr = jnp.where((fmr > 0.5) & valid_r, scr, -1.0)

    posc = (ovc > IOU_THR).astype(jnp.float32)
    fbc = (ovc >= kth).astype(jnp.float32)
    fmc = hp * posc + (1.0 - hp) * fbc
    effc = jnp.where((fmc > 0.5) & valid_c, scc, -1.0)

    npos = jnp.sum((effr > 0.0).astype(jnp.float32))

    racc = jnp.zeros((1, P), jnp.float32)
    for jb in range(NB):
        ej = effc[jb * B:(jb + 1) * B, :]
        ji = jb * B + lax.broadcasted_iota(jnp.int32, (B, 1), 0)
        cmp = (ej > effr) | ((ej == effr) & (ji < lane))
        racc = racc + jnp.sum(cmp.astype(jnp.float32), axis=0, keepdims=True)
    rank = racc

    coords_c = pc_ref[:, 0:4]
    coords_r = pr_ref[0:4, :]
    sr_ref[...] = jnp.zeros((8, P), jnp.float32)
    keep_ref[...] = (lane < npos).astype(jnp.float32)

    lane_b = lax.broadcasted_iota(jnp.int32, (1, B), 1)
    sub_b = lax.broadcasted_iota(jnp.int32, (B, 1), 0)
    ident = (lane_b == sub_b).astype(jnp.float32)

    for rb in range(NB):
        s0 = rb * B

        @pl.when(jnp.float32(s0) < npos)
        def _permute():
            rid = (s0 + lax.broadcasted_iota(jnp.int32, (B, 1), 0)
                   ).astype(jnp.float32)
            ph = (rank == rid).astype(jnp.float32)
            sc_ref[s0:s0 + B, 0:4] = lax.dot_general(
                ph, coords_c, (((1,), (0,)), ((), ())),
                preferred_element_type=jnp.float32,
                precision=lax.Precision.HIGHEST)
            sr_ref[0:4, s0:s0 + B] = lax.dot_general(
                coords_r, ph, (((1,), (1,)), ((), ())),
                preferred_element_type=jnp.float32,
                precision=lax.Precision.HIGHEST)
            sr_ref[4:5, s0:s0 + B] = lax.dot_general(
                effr, ph, (((1,), (1,)), ((), ())),
                preferred_element_type=jnp.float32,
                precision=lax.Precision.HIGHEST)

    for bi in range(NB):
        s0 = bi * B

        @pl.when(jnp.float32(s0) < npos)
        def _process():
            xi1 = sc_ref[s0:s0 + B, 0:1]
            yi1 = sc_ref[s0:s0 + B, 1:2]
            xi2 = sc_ref[s0:s0 + B, 2:3]
            yi2 = sc_ref[s0:s0 + B, 3:4]
            xj1 = sr_ref[0:1, s0:s0 + B]
            yj1 = sr_ref[1:2, s0:s0 + B]
            xj2 = sr_ref[2:3, s0:s0 + B]
            yj2 = sr_ref[3:4, s0:s0 + B]
            iou_d = _iou(xi1, yi1, xi2, yi2, xj1, yj1, xj2, yj2)
            m_ref[...] = ((iou_d > NMS_THR) & (lane_b > sub_b)).astype(
                jnp.float32)

            def body(j, kb):
                mj = m_ref[pl.ds(j, 1), :]
                oh = (lane_b == j).astype(jnp.float32)
                kj = jnp.sum(kb * oh, axis=(0, 1), keepdims=True)
                return kb * (1.0 - mj * kj)

            kb = lax.fori_loop(0, B, body, keep_ref[:, s0:s0 + B])
            keep_ref[:, s0:s0 + B] = kb

            rest = P - (bi + 1) * B
            if rest > 0:
                kbc = lax.dot_general(ident, kb, (((1,), (1,)), ((), ())),
                                      preferred_element_type=jnp.float32,
                                      precision=lax.Precision.HIGHEST)
                t0 = (bi + 1) * B
                xt1 = sr_ref[0:1, t0:t0 + rest]
                yt1 = sr_ref[1:2, t0:t0 + rest]
                xt2 = sr_ref[2:3, t0:t0 + rest]
                yt2 = sr_ref[3:4, t0:t0 + rest]
                iou_x = _iou(xi1, yi1, xi2, yi2, xt1, yt1, xt2, yt2)
                sup = jnp.max((iou_x > NMS_THR).astype(jnp.float32) * kbc,
                              axis=0, keepdims=True)
                keep_ref[:, t0:t0 + rest] = (
                    keep_ref[:, t0:t0 + rest] * (1.0 - sup))

    out_ref[...] = jnp.where(keep_ref[...] > 0.5, sr_ref[...], 0.0)


@jax.jit
def kernel(proposals, gt_bboxes, scores, nms_pre):
    prop = jnp.asarray(proposals, jnp.float32)
    sc = jnp.asarray(scores, jnp.float32)
    prop_p = jnp.concatenate([prop, jnp.zeros((P - N, 4), jnp.float32)], 0)
    sc_p = jnp.concatenate([sc, jnp.zeros((P - N,), jnp.float32)], 0)
    pc = jnp.concatenate(
        [prop_p, sc_p[:, None], jnp.zeros((P, 3), jnp.float32)], 1)
    pr = pc.T
    gt0 = gt_bboxes[0].astype(jnp.float32)
    npre = jnp.asarray(nms_pre, jnp.float32).reshape((1,))

    dets_t = pl.pallas_call(
        _body,
        out_shape=jax.ShapeDtypeStruct((8, P), jnp.float32),
        in_specs=[
            pl.BlockSpec(memory_space=pltpu.SMEM),
            pl.BlockSpec(memory_space=pltpu.SMEM),
            pl.BlockSpec(memory_space=pltpu.VMEM),
            pl.BlockSpec(memory_space=pltpu.VMEM),
        ],
        scratch_shapes=[
            pltpu.VMEM((P, 8), jnp.float32),
            pltpu.VMEM((8, P), jnp.float32),
            pltpu.VMEM((1, P), jnp.float32),
            pltpu.VMEM((B, B), jnp.float32),
        ],
    )(gt0, npre, pr, pc)

    return dets_t.T[:N, :5]
